# baseline (speedup 1.0000x reference)
import functools

import jax
import jax.numpy as jnp
from jax import lax
from jax.experimental import pallas as pl
from jax.experimental.pallas import tpu as pltpu
from jax.experimental.pallas import tpu_sc as plsc

N_ENTITY = 1000000
DIM = 64
BATCH = 16384

_info = plsc.get_sparse_core_info()
_NC = _info.num_cores
_NS = _info.num_subcores
_NW = _NC * _NS
_BPW = BATCH // _NW
_CHUNK = 128
_NCHUNK = _BPW // _CHUNK


def _gather_body(idx_hbm, mu_hbm, out_hbm, idx_v, rows_v, sem):
    wid = lax.axis_index("s") * _NC + lax.axis_index("c")
    base = wid * _BPW
    pltpu.sync_copy(idx_hbm.at[pl.ds(base, _BPW)], idx_v)
    copies = [
        pltpu.async_copy(
            mu_hbm.at[idx_v.at[pl.ds(j * _CHUNK, _CHUNK)]],
            rows_v.at[pl.ds(j * _CHUNK, _CHUNK)],
            sem,
        )
        for j in range(_NCHUNK)
    ]
    for c in copies:
        c.wait()
    pltpu.sync_copy(rows_v, out_hbm.at[pl.ds(base, _BPW)])


@functools.partial(
    pl.kernel,
    out_type=jax.ShapeDtypeStruct((BATCH, DIM), jnp.float32),
    mesh=plsc.VectorSubcoreMesh(core_axis_name="c", subcore_axis_name="s"),
    scratch_types=[
        pltpu.VMEM((_BPW,), jnp.int32),
        pltpu.VMEM((_BPW, DIM), jnp.float32),
        pltpu.SemaphoreType.DMA,
    ],
    compiler_params=pltpu.CompilerParams(use_tc_tiling_on_sc=False),
)
def _sc_gather(idx_hbm, mu_hbm, out_hbm, idx_v, rows_v, sem):
    _gather_body(idx_hbm, mu_hbm, out_hbm, idx_v, rows_v, sem)


def kernel(idx, mu):
    return _sc_gather(idx.astype(jnp.int32), mu)


# speedup vs baseline: 3.0388x; 3.0388x over previous
import functools

import jax
import jax.numpy as jnp
import numpy as np
from jax import lax
from jax.experimental import pallas as pl
from jax.experimental.pallas import tpu as pltpu
from jax.experimental.pallas import tpu_sc as plsc

N_ENTITY = 1000000
DIM = 64
BATCH = 16384

_info = plsc.get_sparse_core_info()
_NC = _info.num_cores
_NS = _info.num_subcores
_NW = _NC * _NS
_BPW = BATCH // _NW
_L = _info.num_lanes
_RING = 8
_GRP = 16


def _extract(blk_slot, l, b, col_v):
    lvec = jnp.full((_L,), l, dtype=jnp.int32)
    bvec = jnp.full((_L,), b, dtype=jnp.int32)
    for k in range(DIM // _L):
        dvec = lax.iota(jnp.int32, _L) + k * _L
        vals = plsc.load_gather(blk_slot, [dvec, lvec])
        plsc.store_scatter(col_v, [dvec, bvec], vals)


def _gather_body(idx_hbm, muT_hbm, outT_hbm, idx_v, blk_v, col_v, lsm, sems):
    wid = lax.axis_index("s") * _NC + lax.axis_index("c")
    base = wid * _BPW
    pltpu.sync_copy(idx_hbm.at[pl.ds(base, _BPW)], idx_v)

    def group(g, _):
        v16 = idx_v[pl.ds(g * _GRP, _GRP)]
        for j in range(_GRP):
            m = g * _GRP + j
            slot = j % _RING
            i = jnp.squeeze(lax.slice(v16, (j,), (j + 1,)))
            c = pl.multiple_of((i >> 7) << 7, 128)
            l = i & 127

            @pl.when(m >= _RING)
            def _():
                pltpu.make_async_copy(
                    muT_hbm.at[:, pl.ds(0, 128)], blk_v.at[slot], sems[slot]
                ).wait()
                _extract(blk_v.at[slot], lsm[slot], m - _RING, col_v)

            pltpu.async_copy(
                muT_hbm.at[:, pl.ds(c, 128)], blk_v.at[slot], sems[slot]
            )
            lsm[slot] = l
        return _

    lax.fori_loop(0, _BPW // _GRP, group, None)
    for jj in range(_RING):
        pltpu.make_async_copy(
            muT_hbm.at[:, pl.ds(0, 128)], blk_v.at[jj], sems[jj]
        ).wait()
        _extract(blk_v.at[jj], lsm[jj], _BPW - _RING + jj, col_v)

    pltpu.sync_copy(col_v, outT_hbm.at[:, pl.ds(base, _BPW)])


@functools.partial(
    pl.kernel,
    out_type=jax.ShapeDtypeStruct((DIM, BATCH), jnp.float32),
    mesh=plsc.VectorSubcoreMesh(core_axis_name="c", subcore_axis_name="s"),
    scratch_types=[
        pltpu.VMEM((_BPW,), jnp.int32),
        pltpu.VMEM((_RING, DIM, 128), jnp.float32),
        pltpu.VMEM((DIM, _BPW), jnp.float32),
        pltpu.SMEM((_RING,), jnp.int32),
    ] + [pltpu.SemaphoreType.DMA] * _RING,
    compiler_params=pltpu.CompilerParams(
        use_tc_tiling_on_sc=True, needs_layout_passes=False
    ),
)
def _sc_gather(idx_hbm, muT_hbm, outT_hbm, idx_v, blk_v, col_v, lsm, *sems):
    _gather_body(idx_hbm, muT_hbm, outT_hbm, idx_v, blk_v, col_v, lsm,
                 list(sems))


def kernel(idx, mu):
    return _sc_gather(idx.astype(jnp.int32), mu.T).T
